# baseline (speedup 1.0000x reference)
import functools
import math

import jax
import jax.numpy as jnp
from jax import lax
from jax.experimental import pallas as pl
from jax.experimental.pallas import tpu as pltpu
from jax.experimental.pallas import tpu_sc as plsc

B = 4096 * 200
D = 64
LANES = 16
NC, NS = 2, 16
NW = NC * NS
BPW = B // NW
C = 1024
NCHUNK = BPW // C
NSTREAM = C // 128
SCALE = math.sqrt(D)

_mesh = plsc.VectorSubcoreMesh(core_axis_name="c", subcore_axis_name="s")


@functools.partial(
    pl.kernel,
    mesh=_mesh,
    out_type=jax.ShapeDtypeStruct((B, D), jnp.float32),
    scratch_types=[
        pltpu.VMEM((NSTREAM, 128), jnp.int32),
        pltpu.VMEM((C, D), jnp.float32),
        pltpu.SemaphoreType.DMA,
    ],
    compiler_params=pltpu.CompilerParams(use_tc_tiling_on_sc=False),
)
def _emb_lookup(x_hbm, table_hbm, out_hbm, idx_v, rows_v, sem):
    wid = lax.axis_index("s") * NC + lax.axis_index("c")
    base = wid * BPW

    def chunk_body(g, carry):
        off = base + g * C
        row_off = pl.multiple_of(off // 128, 8)
        pltpu.sync_copy(x_hbm.at[pl.ds(row_off, NSTREAM)], idx_v)
        for j in range(NSTREAM):
            pltpu.async_copy(
                table_hbm.at[idx_v.at[j]],
                rows_v.at[pl.ds(j * 128, 128)],
                sem,
            )
        for j in range(NSTREAM):
            pltpu.make_async_copy(
                table_hbm.at[idx_v.at[j]],
                rows_v.at[pl.ds(j * 128, 128)],
                sem,
            ).wait()

        def row_body(r, c2):
            for k in range(D // LANES):
                sl = pl.ds(k * LANES, LANES)
                rows_v[r, sl] = rows_v[r, sl] * SCALE
            return c2

        lax.fori_loop(0, C, row_body, 0)

        pltpu.sync_copy(rows_v, out_hbm.at[pl.ds(off, C)])
        return carry

    lax.fori_loop(0, NCHUNK, chunk_body, 0)


def kernel(x, table):
    xf = x.reshape(-1).astype(jnp.int32).reshape(B // 128, 128)
    out = _emb_lookup(xf, table)
    return out.reshape(x.shape[0], x.shape[1], D)


# speedup vs baseline: 1.1055x; 1.1055x over previous
import functools
import math

import jax
import jax.numpy as jnp
from jax import lax
from jax.experimental import pallas as pl
from jax.experimental.pallas import tpu as pltpu
from jax.experimental.pallas import tpu_sc as plsc

NROW, NCOL = 4096, 200
B = NROW * NCOL
D = 64
LANES = 16
NC, NS = 2, 16
NW = NC * NS
RPW = NROW // NW
NR = 4
CH = NR * NCOL
G = RPW // NR
SCALE = math.sqrt(D)

_mesh = plsc.VectorSubcoreMesh(core_axis_name="c", subcore_axis_name="s")


@functools.partial(
    pl.kernel,
    mesh=_mesh,
    out_type=jax.ShapeDtypeStruct((B, D), jnp.float32),
    scratch_types=[
        pltpu.VMEM((2, NR, NCOL), jnp.int32),
        pltpu.VMEM((2, CH, D), jnp.float32),
        pltpu.SemaphoreType.DMA,
        pltpu.SemaphoreType.DMA,
        pltpu.SemaphoreType.DMA,
        pltpu.SemaphoreType.DMA,
    ],
    compiler_params=pltpu.CompilerParams(use_tc_tiling_on_sc=False),
)
def _emb_lookup(x_hbm, table_hbm, out_hbm, idx_v, rows_v, g0, g1, o0, o1):
    wid = lax.axis_index("s") * NC + lax.axis_index("c")
    row_base = wid * RPW
    gsem = (g0, g1)
    osem = (o0, o1)

    def stage(g, b):
        pltpu.sync_copy(x_hbm.at[pl.ds(row_base + g * NR, NR)], idx_v.at[b])
        for r in range(NR):
            pltpu.async_copy(
                table_hbm.at[idx_v.at[b, r, pl.ds(0, 128)]],
                rows_v.at[b, pl.ds(r * NCOL, 128)],
                gsem[b],
            )
            pltpu.async_copy(
                table_hbm.at[idx_v.at[b, r, pl.ds(128, NCOL - 128)]],
                rows_v.at[b, pl.ds(r * NCOL + 128, NCOL - 128)],
                gsem[b],
            )

    def wait_gathers(b):
        for r in range(NR):
            pltpu.make_async_copy(
                table_hbm.at[idx_v.at[b, r, pl.ds(0, 128)]],
                rows_v.at[b, pl.ds(r * NCOL, 128)],
                gsem[b],
            ).wait()
            pltpu.make_async_copy(
                table_hbm.at[idx_v.at[b, r, pl.ds(128, NCOL - 128)]],
                rows_v.at[b, pl.ds(r * NCOL + 128, NCOL - 128)],
                gsem[b],
            ).wait()

    def wait_writeback(b):
        pltpu.make_async_copy(
            rows_v.at[b], out_hbm.at[pl.ds(0, CH)], osem[b]
        ).wait()

    stage(0, 0)

    def pair_body(k, carry):
        for b in (0, 1):
            gc = 2 * k + b
            nxt = gc + 1

            @pl.when(nxt < G)
            def _():
                @pl.when(nxt >= 2)
                def _():
                    wait_writeback(1 - b)

                stage(nxt, 1 - b)

            wait_gathers(b)

            @plsc.parallel_loop(0, CH, 1, unroll=8)
            def _(r):
                for kk in range(D // LANES):
                    sl = pl.ds(kk * LANES, LANES)
                    rows_v[b, r, sl] = rows_v[b, r, sl] * SCALE

            pltpu.async_copy(
                rows_v.at[b],
                out_hbm.at[pl.ds((row_base + gc * NR) * NCOL, CH)],
                osem[b],
            )
        return carry

    lax.fori_loop(0, G // 2, pair_body, 0)
    wait_writeback(0)
    wait_writeback(1)


def kernel(x, table):
    out = _emb_lookup(x.astype(jnp.int32), table)
    return out.reshape(NROW, NCOL, D)
